# baseline (speedup 1.0000x reference)
import functools
import jax
import jax.numpy as jnp
from jax import lax
from jax.experimental import pallas as pl
from jax.experimental.pallas import tpu as pltpu
from jax.experimental.pallas import tpu_sc as plsc

B, N, F = 4, 8192, 2048
NW = 32
WPB = NW // B
FW = F // WPB
R = 64
NT = N // R
NV = FW // 16

_mesh = plsc.VectorSubcoreMesh(core_axis_name="c", subcore_axis_name="s")


@functools.partial(
    pl.kernel,
    mesh=_mesh,
    out_type=jax.ShapeDtypeStruct((B, N, F), jnp.float32),
    scratch_types=[
        pltpu.VMEM((2, R, FW), jnp.float32),
        pltpu.VMEM_SHARED((16, 2, R, FW), jnp.float32),
        pltpu.VMEM_SHARED((16, 2, R, FW), jnp.float32),
        pltpu.SemaphoreType.DMA,
        pltpu.SemaphoreType.DMA,
        pltpu.SemaphoreType.DMA,
        pltpu.SemaphoreType.DMA,
        pltpu.SemaphoreType.DMA,
        pltpu.SemaphoreType.DMA,
        pltpu.SemaphoreType.DMA,
        pltpu.SemaphoreType.DMA,
    ],
)
def _sc_cumsum(x_hbm, out_hbm, cbuf, in_sh, out_sh, *sems):
    wid = lax.axis_index("s") * 2 + lax.axis_index("c")
    sid = lax.axis_index("s")
    b = wid // WPB
    f0 = (wid % WPB) * FW
    h2s_sem = sems[0:2]
    s2t_sem = sems[2:4]
    t2s_sem = sems[4:6]
    s2h_sem = sems[6:8]

    def h2s(t, s):
        return pltpu.make_async_copy(
            x_hbm.at[b, pl.ds(t * R, R), pl.ds(f0, FW)],
            in_sh.at[sid, s],
            h2s_sem[s],
        )

    def s2t(s):
        return pltpu.make_async_copy(in_sh.at[sid, s], cbuf.at[s], s2t_sem[s])

    def t2s(s):
        return pltpu.make_async_copy(cbuf.at[s], out_sh.at[sid, s], t2s_sem[s])

    def s2h(t, s):
        return pltpu.make_async_copy(
            out_sh.at[sid, s],
            out_hbm.at[b, pl.ds(t * R, R), pl.ds(f0, FW)],
            s2h_sem[s],
        )

    h2s(0, 0).start()
    h2s(1, 1).start()
    h2s(0, 0).wait()
    s2t(0).start()

    def phase(t, s, carry):
        so = 1 - s
        s2t(s).wait()

        @pl.when(t + 2 < NT)
        def _():
            h2s(t + 2, s).start()

        @pl.when(t >= 1)
        def _():
            t2s(so).wait()
            s2h(t - 1, so).start()

        @pl.when(t + 1 < NT)
        def _():
            h2s(t + 1, so).wait()
            s2t(so).start()

        def row(r, acc):
            new = []
            for j in range(NV):
                v = acc[j] + cbuf[s, r, pl.ds(16 * j, 16)]
                cbuf[s, r, pl.ds(16 * j, 16)] = v
                new.append(v)
            return tuple(new)

        carry = lax.fori_loop(0, R, row, carry, unroll=2)

        @pl.when(t >= 2)
        def _():
            s2h(t - 2, s).wait()

        t2s(s).start()
        return carry

    def two(i, carry):
        t = i * 2
        carry = phase(t, 0, carry)
        carry = phase(t + 1, 1, carry)
        return carry

    zeros = tuple(jnp.zeros((16,), jnp.float32) for _ in range(NV))
    lax.fori_loop(0, NT // 2, two, zeros)

    t2s(1).wait()
    s2h(NT - 1, 1).start()
    s2h(NT - 2, 0).wait()
    s2h(NT - 1, 1).wait()


def kernel(x):
    return _sc_cumsum(x)


# speedup vs baseline: 1.8235x; 1.8235x over previous
import functools
import jax
import jax.numpy as jnp
from jax import lax
from jax.experimental import pallas as pl
from jax.experimental.pallas import tpu as pltpu
from jax.experimental.pallas import tpu_sc as plsc

B, N, F = 4, 8192, 2048
NW = 32
WPB = NW // B
FW = F // WPB
R = 64
NT = N // R
NV = FW // 16
NS = 4
LD = 2

_mesh = plsc.VectorSubcoreMesh(core_axis_name="c", subcore_axis_name="s")


@functools.partial(
    pl.kernel,
    mesh=_mesh,
    out_type=jax.ShapeDtypeStruct((B, N, F), jnp.float32),
    scratch_types=[
        pltpu.VMEM((NS, R, FW), jnp.float32),
        pltpu.SemaphoreType.DMA,
        pltpu.SemaphoreType.DMA,
        pltpu.SemaphoreType.DMA,
        pltpu.SemaphoreType.DMA,
        pltpu.SemaphoreType.DMA,
        pltpu.SemaphoreType.DMA,
        pltpu.SemaphoreType.DMA,
        pltpu.SemaphoreType.DMA,
    ],
)
def _sc_cumsum(x_hbm, out_hbm, buf, *sems):
    wid = lax.axis_index("s") * 2 + lax.axis_index("c")
    b = wid // WPB
    f0 = (wid % WPB) * FW
    lsems = sems[:NS]
    ssems = sems[NS:]

    def load_copy(t, s):
        return pltpu.make_async_copy(
            x_hbm.at[b, pl.ds(t * R, R), pl.ds(f0, FW)],
            buf.at[s],
            lsems[s],
        )

    def store_copy(t, s):
        return pltpu.make_async_copy(
            buf.at[s],
            out_hbm.at[b, pl.ds(t * R, R), pl.ds(f0, FW)],
            ssems[s],
        )

    for k in range(LD):
        load_copy(k, k).start()

    def phase(t, s, carry):
        load_copy(t, s).wait()

        sl = (s + LD) % NS

        @pl.when(t + LD < NT)
        def _():
            @pl.when(t >= NS - LD)
            def _():
                store_copy(t - (NS - LD), sl).wait()

            load_copy(t + LD, sl).start()

        def row(r, acc):
            new = []
            for j in range(NV):
                v = acc[j] + buf[s, r, pl.ds(16 * j, 16)]
                buf[s, r, pl.ds(16 * j, 16)] = v
                new.append(v)
            return tuple(new)

        carry = lax.fori_loop(0, R, row, carry, unroll=2)
        store_copy(t, s).start()
        return carry

    def ring(i, carry):
        t = i * NS
        for k in range(NS):
            carry = phase(t + k, k, carry)
        return carry

    zeros = tuple(jnp.zeros((16,), jnp.float32) for _ in range(NV))
    lax.fori_loop(0, NT // NS, ring, zeros)

    for t in range(NT - (NS - LD), NT):
        store_copy(t, t % NS).wait()


def kernel(x):
    return _sc_cumsum(x)
